# Initial kernel scaffold; baseline (speedup 1.0000x reference)
import jax
import jax.numpy as jnp
from jax.experimental import pallas as pl


def kernel(x, Wr, br, Wn, bn, W1, b1, W2, b2):
    raise NotImplementedError("write your pallas kernel here")


# speedup vs baseline: 1.1235x; 1.1235x over previous
import functools

import jax
import jax.numpy as jnp
import numpy as np
from jax.experimental import pallas as pl
from jax.experimental.pallas import tpu as pltpu

N_EMBED = 384
NUM_EXPERTS = 8
HIDDEN = 4 * N_EMBED
BS = 8
HW = 196
N_TOK = BS * HW


def _router_body(flatx_ref, wr_ref, br_ref, xavg_ref, wn_ref, bn_ref,
                 noise_ref, gate_ref, msk_ref):
    logits = jnp.dot(flatx_ref[...], wr_ref[...],
                     preferred_element_type=jnp.float32) + br_ref[...]
    nl = jnp.dot(xavg_ref[...], wn_ref[...],
                 preferred_element_type=jnp.float32) + bn_ref[...]
    scale = jax.nn.softplus(nl)
    scale_tok = jnp.concatenate(
        [jnp.broadcast_to(scale[b:b + 1, :], (HW, NUM_EXPERTS))
         for b in range(BS)], axis=0)
    noisy = logits + noise_ref[...] * scale_tok
    m1 = jnp.max(noisy, axis=1, keepdims=True)
    i1 = jnp.argmax(noisy, axis=1)
    lane = jax.lax.broadcasted_iota(jnp.int32, noisy.shape, 1)
    oh1 = (lane == i1[:, None]).astype(jnp.float32)
    masked = jnp.where(lane == i1[:, None], -jnp.inf, noisy)
    m2 = jnp.max(masked, axis=1, keepdims=True)
    i2 = jnp.argmax(masked, axis=1)
    oh2 = (lane == i2[:, None]).astype(jnp.float32)
    z = jnp.exp(m2 - m1)
    g1 = 1.0 / (1.0 + z)
    g2 = z / (1.0 + z)
    gate_ref[...] = g1 * oh1 + g2 * oh2
    msk_ref[...] = oh1 + oh2


def _router(flat_x, Wr, br, x_avg, Wn, bn, noise):
    return pl.pallas_call(
        _router_body,
        in_specs=[pl.BlockSpec(a.shape, lambda: (0,) * a.ndim)
                  for a in (flat_x, Wr, br, x_avg, Wn, bn, noise)],
        out_specs=[
            pl.BlockSpec((N_TOK, NUM_EXPERTS), lambda: (0, 0)),
            pl.BlockSpec((N_TOK, NUM_EXPERTS), lambda: (0, 0)),
        ],
        out_shape=[
            jax.ShapeDtypeStruct((N_TOK, NUM_EXPERTS), jnp.float32),
            jax.ShapeDtypeStruct((N_TOK, NUM_EXPERTS), jnp.float32),
        ],
    )(flat_x, Wr, br, x_avg, Wn, bn, noise)


def _moe_body(flatx_ref, gate_ref, msk_ref, w1_ref, b1_ref, w2_ref, b2_ref,
              out_ref):
    e = pl.program_id(0)
    h1 = jnp.maximum(
        jnp.dot(flatx_ref[...], w1_ref[0],
                preferred_element_type=jnp.float32) + b1_ref[0], 0.0)
    out_e = jnp.dot(h1, w2_ref[0],
                    preferred_element_type=jnp.float32) + b2_ref[0]
    lane_e = jax.lax.broadcasted_iota(jnp.int32, (N_TOK, NUM_EXPERTS), 1)
    sel = jnp.sum(jnp.where(lane_e == e, msk_ref[...], 0.0),
                  axis=1, keepdims=True)
    gcol = jnp.sum(jnp.where(lane_e == e, gate_ref[...], 0.0),
                   axis=1, keepdims=True)
    prev = jnp.where(e == 0, jnp.zeros_like(out_ref[...]), out_ref[...])
    out_ref[...] = jnp.where(sel > 0.0, out_e * gcol, prev)


def _moe(flat_x, gate, msk, W1, b1, W2, b2):
    return pl.pallas_call(
        _moe_body,
        grid=(NUM_EXPERTS,),
        in_specs=[
            pl.BlockSpec((N_TOK, N_EMBED), lambda e: (0, 0)),
            pl.BlockSpec((N_TOK, NUM_EXPERTS), lambda e: (0, 0)),
            pl.BlockSpec((N_TOK, NUM_EXPERTS), lambda e: (0, 0)),
            pl.BlockSpec((1, N_EMBED, HIDDEN), lambda e: (e, 0, 0)),
            pl.BlockSpec((1, 1, HIDDEN), lambda e: (e, 0, 0)),
            pl.BlockSpec((1, HIDDEN, N_EMBED), lambda e: (e, 0, 0)),
            pl.BlockSpec((1, 1, N_EMBED), lambda e: (e, 0, 0)),
        ],
        out_specs=pl.BlockSpec((N_TOK, N_EMBED), lambda e: (0, 0)),
        out_shape=jax.ShapeDtypeStruct((N_TOK, N_EMBED), jnp.float32),
    )(flat_x, gate, msk, W1, b1, W2, b2)


def kernel(x, Wr, br, Wn, bn, W1, b1, W2, b2):
    bs, dim, h, w = x.shape
    flat_x = x.reshape(bs * h * w, dim)
    x_avg = jnp.mean(x, axis=(2, 3))
    noise = jax.random.normal(jax.random.key(42), (bs, h * w, NUM_EXPERTS),
                              dtype=jnp.float32).reshape(bs * h * w, NUM_EXPERTS)
    gate, msk = _router(flat_x, Wr, br.reshape(1, -1), x_avg, Wn,
                        bn.reshape(1, -1), noise)
    out = _moe(flat_x, gate, msk, W1, b1.reshape(NUM_EXPERTS, 1, HIDDEN),
               W2, b2.reshape(NUM_EXPERTS, 1, N_EMBED))
    return out.reshape(bs, dim, h, w)


# speedup vs baseline: 1.1302x; 1.0060x over previous
import functools

import jax
import jax.numpy as jnp
import numpy as np
from jax.experimental import pallas as pl
from jax.experimental.pallas import tpu as pltpu

N_EMBED = 384
NUM_EXPERTS = 8
HIDDEN = 4 * N_EMBED
BS = 8
HW = 196
N_TOK = BS * HW
BLK = 128
NBLK = 20


def _router_body(flatx_ref, wr_ref, br_ref, xavg_ref, wn_ref, bn_ref,
                 noise_ref, pos_ref, g_ref, be_ref):
    logits = jnp.dot(flatx_ref[...], wr_ref[...],
                     preferred_element_type=jnp.float32) + br_ref[...]
    nl = jnp.dot(xavg_ref[...], wn_ref[...],
                 preferred_element_type=jnp.float32) + bn_ref[...]
    scale = jax.nn.softplus(nl)
    scale_tok = jnp.concatenate(
        [jnp.broadcast_to(scale[b:b + 1, :], (HW, NUM_EXPERTS))
         for b in range(BS)], axis=0)
    noisy = logits + noise_ref[...] * scale_tok
    m1 = jnp.max(noisy, axis=1, keepdims=True)
    i1 = jnp.argmax(noisy, axis=1)
    lane = jax.lax.broadcasted_iota(jnp.int32, noisy.shape, 1)
    masked = jnp.where(lane == i1[:, None], -jnp.inf, noisy)
    m2 = jnp.max(masked, axis=1, keepdims=True)
    i2 = jnp.argmax(masked, axis=1)
    z = jnp.exp(m2 - m1)
    g1 = 1.0 / (1.0 + z)
    g2 = z / (1.0 + z)
    g_ref[...] = jnp.where((i1 > i2)[:, None], g1, g2)
    estar = jnp.maximum(i1, i2)

    M = (lane == estar[:, None]).astype(jnp.float32)
    r_i = jax.lax.broadcasted_iota(jnp.int32, (N_TOK, N_TOK), 0)
    c_j = jax.lax.broadcasted_iota(jnp.int32, (N_TOK, N_TOK), 1)
    L = (c_j < r_i).astype(jnp.float32)
    prior = jnp.dot(L, M, preferred_element_type=jnp.float32)
    rank = jnp.sum(prior * M, axis=1, keepdims=True)
    counts = jnp.sum(M, axis=0, keepdims=True)
    nb = jnp.ceil(counts * (1.0 / BLK))
    padded = nb * BLK
    ii = jax.lax.broadcasted_iota(jnp.int32, (NUM_EXPERTS, NUM_EXPERTS), 0)
    jj = jax.lax.broadcasted_iota(jnp.int32, (NUM_EXPERTS, NUM_EXPERTS), 1)
    U = (ii < jj).astype(jnp.float32)
    pstart = jnp.dot(padded, U, preferred_element_type=jnp.float32)
    pstart_tok = jnp.sum(pstart * M, axis=1, keepdims=True)
    pos_ref[...] = (pstart_tok + rank).astype(jnp.int32)
    ends = pstart * (1.0 / BLK) + nb
    total = jnp.sum(nb, axis=1, keepdims=True)
    bcol = jax.lax.broadcasted_iota(jnp.int32, (NBLK, 1), 0).astype(jnp.float32)
    bc = jnp.minimum(bcol, total - 1.0)
    be_ref[...] = jnp.sum((ends <= bc).astype(jnp.float32), axis=1,
                          keepdims=True).astype(jnp.int32)


def _router(flat_x, Wr, br, x_avg, Wn, bn, noise):
    return pl.pallas_call(
        _router_body,
        in_specs=[pl.BlockSpec(a.shape, lambda: (0,) * a.ndim)
                  for a in (flat_x, Wr, br, x_avg, Wn, bn, noise)],
        out_specs=[
            pl.BlockSpec((N_TOK, 1), lambda: (0, 0)),
            pl.BlockSpec((N_TOK, 1), lambda: (0, 0)),
            pl.BlockSpec((NBLK, 1), lambda: (0, 0)),
        ],
        out_shape=[
            jax.ShapeDtypeStruct((N_TOK, 1), jnp.int32),
            jax.ShapeDtypeStruct((N_TOK, 1), jnp.float32),
            jax.ShapeDtypeStruct((NBLK, 1), jnp.int32),
        ],
    )(flat_x, Wr, br, x_avg, Wn, bn, noise)


def _ffn_body(be_ref, pos_ref, g_ref, flatx_ref, w1_ref, b1_ref, w2_ref,
              b2_ref, out_ref):
    b = pl.program_id(0)
    slot = b * BLK + jax.lax.broadcasted_iota(jnp.int32, (BLK, 1), 0)
    P = (pos_ref[...] == slot).astype(jnp.float32)
    xg = jnp.dot(P, flatx_ref[...], preferred_element_type=jnp.float32)
    h1 = jnp.maximum(
        jnp.dot(xg, w1_ref[0], preferred_element_type=jnp.float32)
        + b1_ref[0], 0.0)
    o = jnp.dot(h1, w2_ref[0], preferred_element_type=jnp.float32) + b2_ref[0]
    gblk = jnp.sum(P * g_ref[...], axis=1, keepdims=True)
    contrib = jax.lax.dot_general(P, o * gblk, (((0,), (0,)), ((), ())),
                                  preferred_element_type=jnp.float32)
    prev = jnp.where(b == 0, 0.0, out_ref[...])
    out_ref[...] = prev + contrib


def _ffn(be, pos_row, g_row, flat_x, W1, b1, W2, b2):
    grid_spec = pltpu.PrefetchScalarGridSpec(
        num_scalar_prefetch=1,
        grid=(NBLK,),
        in_specs=[
            pl.BlockSpec((1, N_TOK), lambda b, be: (0, 0)),
            pl.BlockSpec((1, N_TOK), lambda b, be: (0, 0)),
            pl.BlockSpec((N_TOK, N_EMBED), lambda b, be: (0, 0)),
            pl.BlockSpec((1, N_EMBED, HIDDEN), lambda b, be: (be[b], 0, 0)),
            pl.BlockSpec((1, 1, HIDDEN), lambda b, be: (be[b], 0, 0)),
            pl.BlockSpec((1, HIDDEN, N_EMBED), lambda b, be: (be[b], 0, 0)),
            pl.BlockSpec((1, 1, N_EMBED), lambda b, be: (be[b], 0, 0)),
        ],
        out_specs=pl.BlockSpec((N_TOK, N_EMBED), lambda b, be: (0, 0)),
    )
    return pl.pallas_call(
        _ffn_body,
        grid_spec=grid_spec,
        out_shape=jax.ShapeDtypeStruct((N_TOK, N_EMBED), jnp.float32),
    )(be, pos_row, g_row, flat_x, W1, b1, W2, b2)


def kernel(x, Wr, br, Wn, bn, W1, b1, W2, b2):
    bs, dim, h, w = x.shape
    flat_x = x.reshape(bs * h * w, dim)
    x_avg = jnp.mean(x, axis=(2, 3))
    noise = jax.random.normal(jax.random.key(42), (bs, h * w, NUM_EXPERTS),
                              dtype=jnp.float32).reshape(bs * h * w, NUM_EXPERTS)
    pos, g, be = _router(flat_x, Wr, br.reshape(1, -1), x_avg, Wn,
                         bn.reshape(1, -1), noise)
    out = _ffn(be.reshape(NBLK), pos.reshape(1, N_TOK), g.reshape(1, N_TOK),
               flat_x, W1, b1.reshape(NUM_EXPERTS, 1, HIDDEN),
               W2, b2.reshape(NUM_EXPERTS, 1, N_EMBED))
    return out.reshape(bs, dim, h, w)
